# baseline (speedup 1.0000x reference)
import dataclasses
import functools

import jax
import jax.numpy as jnp
from jax import lax
from jax.experimental import pallas as pl
from jax.experimental.pallas import tpu as pltpu
from jax.experimental.pallas import tpu_sc as plsc

N = 10000
FEAT = 256
HALF = 128
NOUT = 128
E = 160000

EB = 128
NBLK = 1280
EP = NBLK * EB
NSUB = 16
BPS = NBLK // NSUB
A1_ROWS = 10112
RED = 640

PHALF = N // 2
PACC = PHALF + 8
EBP = 64
CAP_BLK = 1408
CAP = CAP_BLK * EBP
QMAX = CAP_BLK // NSUB
QCH = QMAX // 2
RPP = 312
PTAIL = PHALF - NSUB * RPP

BM = 400
GRID = N // BM

_PREC = lax.Precision.DEFAULT

_mesh = plsc.VectorSubcoreMesh(core_axis_name="c", subcore_axis_name="s")

_sc_params = pltpu.CompilerParams()
if "needs_layout_passes" in pltpu.CompilerParams.__dataclass_fields__:
    _sc_params = dataclasses.replace(_sc_params, needs_layout_passes=False)


def _agg1_body(sigp_hbm, srcb_hbm, dstb_hbm, out_hbm,
               sig_v, idx3s, idx3d, acc_v, red_v, outacc, sh, sem):
    cid = lax.axis_index("c")
    sid = lax.axis_index("s")

    @pl.when(cid == 0)
    def _():
        pltpu.sync_copy(sigp_hbm, sig_v)
        pltpu.sync_copy(srcb_hbm.at[pl.ds(sid * BPS, BPS)], idx3s)
        pltpu.sync_copy(dstb_hbm.at[pl.ds(sid * BPS, BPS)], idx3d)

        @pl.loop(0, A1_ROWS // 16)
        def _(i):
            acc_v[pl.ds(i * 16, 16)] = jnp.zeros((16,), jnp.float32)

        @pl.loop(0, BPS)
        def _(j):
            @pl.loop(0, EB // 16)
            def _(k):
                sv = idx3s[j, 0, pl.ds(k * 16, 16)]
                dv = idx3d[j, 0, pl.ds(k * 16, 16)]
                vals = plsc.load_gather(sig_v, [sv])
                plsc.addupdate_scatter(acc_v, [dv], vals)

        pltpu.sync_copy(acc_v, sh.at[sid, 0])
        plsc.subcore_barrier()

        c0 = jnp.minimum(sid * RED, A1_ROWS - RED)
        pltpu.sync_copy(sh.at[:, :, pl.ds(c0, RED)], red_v)

        @pl.loop(0, RED // 16)
        def _(i):
            v = sig_v[pl.ds(c0 + i * 16, 16)]
            for k in range(NSUB):
                v = v + red_v[k, 0, pl.ds(i * 16, 16)]
            outacc[pl.ds(i * 16, 16)] = v

        pltpu.sync_copy(outacc, out_hbm.at[pl.ds(c0, RED)])


@functools.partial(
    pl.kernel,
    mesh=_mesh,
    out_type=jax.ShapeDtypeStruct((A1_ROWS,), jnp.float32),
    scratch_types=[
        pltpu.VMEM((A1_ROWS,), jnp.float32),
        pltpu.VMEM((BPS, 1, EB), jnp.int32),
        pltpu.VMEM((BPS, 1, EB), jnp.int32),
        pltpu.VMEM((A1_ROWS,), jnp.float32),
        pltpu.VMEM((NSUB, 1, RED), jnp.float32),
        pltpu.VMEM((RED,), jnp.float32),
        pltpu.VMEM_SHARED((NSUB, 1, A1_ROWS), jnp.float32),
        pltpu.SemaphoreType.DMA,
    ],
    compiler_params=_sc_params,
)
def _agg1(*args):
    _agg1_body(*args)


def _aggp_body(x3_hbm, losrc, lodst, hisrc, hidst, cnt2_hbm, out_hbm,
               srcv, dstv, cntv, rowsbuf, acc, semg0, semg1):
    cid = lax.axis_index("c")
    sid = lax.axis_index("s")
    base = cid * PHALF
    semg = (semg0, semg1)

    pltpu.sync_copy(x3_hbm.at[pl.ds(base + sid * RPP, RPP)],
                    acc.at[pl.ds(sid * RPP, RPP)])

    @pl.when(sid == NSUB - 1)
    def _():
        pltpu.sync_copy(x3_hbm.at[pl.ds(base + NSUB * RPP, PTAIL)],
                        acc.at[pl.ds(NSUB * RPP, PTAIL)])

    pltpu.sync_copy(cnt2_hbm, cntv)
    nb0 = jnp.max(cntv[0, pl.ds(0, 16)])
    nb1 = jnp.max(cntv[1, pl.ds(0, 16)])
    nb = jnp.where(cid == 0, nb0, nb1)
    quota = (nb + NSUB - 1) // NSUB
    s0 = sid * quota
    myn = jnp.clip(nb - s0, 0, quota)

    plsc.subcore_barrier()

    def g_slot(hp):
        return rowsbuf.at[pl.ds(hp * EBP, EBP)]

    def run_class(sb, db):
        for ch in range(2):
            off0 = s0 + ch * QCH
            n_ch = jnp.clip(myn - ch * QCH, 0, QCH)

            @pl.when(n_ch > 0)
            def _():
                pltpu.sync_copy(sb.at[pl.ds(off0, QCH)], srcv)
                pltpu.sync_copy(db.at[pl.ds(off0, QCH)], dstv)

                pltpu.make_async_copy(x3_hbm.at[srcv.at[0, 0]], g_slot(0),
                                      semg[0]).start()

                @pl.when(n_ch > 1)
                def _():
                    pltpu.make_async_copy(x3_hbm.at[srcv.at[1, 0]], g_slot(1),
                                          semg[1]).start()

                @pl.loop(0, (n_ch + 1) // 2)
                def _(g2):
                    for hp in range(2):
                        j = g2 * 2 + hp

                        @pl.when(j < n_ch)
                        def _():
                            pltpu.make_async_copy(x3_hbm.at[srcv.at[0, 0]],
                                                  g_slot(hp), semg[hp]).wait()

                            @pl.when(j + 2 < n_ch)
                            def _():
                                pltpu.make_async_copy(
                                    x3_hbm.at[srcv.at[j + 2, 0]],
                                    g_slot(hp), semg[hp]).start()

                            pltpu.sync_copy(g_slot(hp),
                                            acc.at[dstv.at[j, 0]], add=True)

    @pl.when(cid == 0)
    def _():
        run_class(losrc, lodst)

    @pl.when(cid == 1)
    def _():
        run_class(hisrc, hidst)

    plsc.subcore_barrier()

    pltpu.sync_copy(acc.at[pl.ds(sid * RPP, RPP)],
                    out_hbm.at[pl.ds(base + sid * RPP, RPP)])

    @pl.when(sid == NSUB - 1)
    def _():
        pltpu.sync_copy(acc.at[pl.ds(NSUB * RPP, PTAIL)],
                        out_hbm.at[pl.ds(base + NSUB * RPP, PTAIL)])


@functools.partial(
    pl.kernel,
    mesh=_mesh,
    out_type=jax.ShapeDtypeStruct((N, 2, HALF), jnp.float32),
    scratch_types=[
        pltpu.VMEM((QCH, 1, EBP), jnp.int32),
        pltpu.VMEM((QCH, 1, EBP), jnp.int32),
        pltpu.VMEM((2, 16), jnp.int32),
        pltpu.VMEM((2 * EBP, 2, HALF), jnp.float32),
        pltpu.VMEM_SHARED((PACC, 2, HALF), jnp.float32),
        pltpu.SemaphoreType.DMA,
        pltpu.SemaphoreType.DMA,
    ],
    compiler_params=_sc_params,
)
def _aggp(*args):
    _aggp_body(*args)


def _stats_body(t_ref, w0_ref, b0_ref, stats_ref):
    i = pl.program_id(0)
    xp = jnp.maximum(t_ref[...] * w0_ref[...] + b0_ref[...], 0.0)

    @pl.when(i == 0)
    def _():
        stats_ref[...] = jnp.zeros_like(stats_ref)

    stats_ref[0:1, :] += jnp.sum(xp, axis=0, keepdims=True)
    stats_ref[1:2, :] += jnp.sum(xp * xp, axis=0, keepdims=True)


def _norm_body(t_ref, stats_ref, w0_ref, b0_ref, g_ref, be_ref, al_ref, out_ref):
    xp = jnp.maximum(t_ref[...] * w0_ref[...] + b0_ref[...], 0.0)
    m = stats_ref[0:1, :] * (1.0 / N)
    ex2 = stats_ref[1:2, :] * (1.0 / N)
    al = al_ref[...]
    var = ex2 - 2.0 * al * m * m + al * al * m * m
    y = g_ref[...] * ((xp - al * m) * lax.rsqrt(var + 1e-5)) + be_ref[...]
    out_ref[...] = y.reshape(BM, 2, HALF)


def _mlp_body(h_ref, x_ref, w1_ref, b1_ref, w2_ref, b2_ref, out_ref):
    h = h_ref[...].reshape(BM, FEAT)
    z = jnp.maximum(
        jnp.dot(h, w1_ref[...], preferred_element_type=jnp.float32,
                precision=_PREC) + b1_ref[...], 0.0)
    y = jnp.dot(z, w2_ref[...], preferred_element_type=jnp.float32,
                precision=_PREC) + b2_ref[...]
    x = x_ref[...].reshape(BM, FEAT)
    xn = x + jnp.maximum(y, 0.0)
    out_ref[...] = xn.reshape(BM, 2, HALF)


def _final_body(x_ref, wf_ref, bf_ref, out_ref):
    x = x_ref[...].reshape(BM, FEAT)
    out_ref[...] = jnp.dot(x, wf_ref[...], preferred_element_type=jnp.float32,
                           precision=_PREC) + bf_ref[...]


def _row_spec(shape):
    return pl.BlockSpec(shape, lambda i: tuple(0 for _ in shape))


def _tc_stats(t1, w0row, b0row):
    return pl.pallas_call(
        _stats_body,
        grid=(GRID,),
        in_specs=[
            pl.BlockSpec((BM, 1), lambda i: (i, 0)),
            _row_spec((1, FEAT)),
            _row_spec((1, FEAT)),
        ],
        out_specs=pl.BlockSpec((8, FEAT), lambda i: (0, 0)),
        out_shape=jax.ShapeDtypeStruct((8, FEAT), jnp.float32),
    )(t1, w0row, b0row)


def _tc_norm(t1, stats, w0row, b0row, grow, berow, alrow):
    return pl.pallas_call(
        _norm_body,
        grid=(GRID,),
        in_specs=[
            pl.BlockSpec((BM, 1), lambda i: (i, 0)),
            _row_spec((8, FEAT)),
            _row_spec((1, FEAT)),
            _row_spec((1, FEAT)),
            _row_spec((1, FEAT)),
            _row_spec((1, FEAT)),
            _row_spec((1, FEAT)),
        ],
        out_specs=pl.BlockSpec((BM, 2, HALF), lambda i: (i, 0, 0)),
        out_shape=jax.ShapeDtypeStruct((N, 2, HALF), jnp.float32),
    )(t1, stats, w0row, b0row, grow, berow, alrow)


def _tc_mlp(h3, x3, w1t, b1row, w2t, b2row):
    return pl.pallas_call(
        _mlp_body,
        grid=(GRID,),
        in_specs=[
            pl.BlockSpec((BM, 2, HALF), lambda i: (i, 0, 0)),
            pl.BlockSpec((BM, 2, HALF), lambda i: (i, 0, 0)),
            _row_spec((FEAT, FEAT)),
            _row_spec((1, FEAT)),
            _row_spec((FEAT, FEAT)),
            _row_spec((1, FEAT)),
        ],
        out_specs=pl.BlockSpec((BM, 2, HALF), lambda i: (i, 0, 0)),
        out_shape=jax.ShapeDtypeStruct((N, 2, HALF), jnp.float32),
    )(h3, x3, w1t, b1row, w2t, b2row)


def _tc_final(x3, wft, bfrow):
    return pl.pallas_call(
        _final_body,
        grid=(GRID,),
        in_specs=[
            pl.BlockSpec((BM, 2, HALF), lambda i: (i, 0, 0)),
            _row_spec((FEAT, NOUT)),
            _row_spec((1, NOUT)),
        ],
        out_specs=pl.BlockSpec((BM, NOUT), lambda i: (i, 0)),
        out_shape=jax.ShapeDtypeStruct((N, NOUT), jnp.float32),
    )(x3, wft, bfrow)


def kernel(signals, edge_index, W0, b0, W1_0, b1_0, W2_0, b2_0, W1_1, b1_1,
           W2_1, W1_2, b1_2, W2_2, gn_gamma, gn_beta, gn_alpha, Wf, bf):
    src = edge_index[0].astype(jnp.int32)
    dst = edge_index[1].astype(jnp.int32)
    pad = EP - E
    srcp = jnp.concatenate([src, jnp.zeros((pad,), jnp.int32)]).reshape(NBLK, 1, EB)
    dstp = jnp.concatenate([dst, jnp.full((pad,), N, jnp.int32)]).reshape(NBLK, 1, EB)

    key = (dst >= PHALF).astype(jnp.int32)
    sv = (src << 14) | dst
    cum_lo = jnp.cumsum(1 - key)
    cum_hi = jnp.cumsum(key)
    nlo = cum_lo[-1]
    nhi = E - nlo
    lo_pos = jnp.where(key == 0, cum_lo - 1, CAP)
    hi_pos = jnp.where(key == 1, cum_hi - 1, CAP)
    pad_lo = PHALF
    pad_hi = N
    lo_sv = jnp.full((CAP,), pad_lo, jnp.int32).at[lo_pos].set(sv, mode="drop")
    hi_sv = jnp.full((CAP,), pad_hi, jnp.int32).at[hi_pos].set(sv, mode="drop")
    lo_src = (lo_sv >> 14).reshape(CAP_BLK, 1, EBP)
    lo_dst = (lo_sv & 16383).reshape(CAP_BLK, 1, EBP)
    hi_src = (hi_sv >> 14).reshape(CAP_BLK, 1, EBP)
    hi_dst = ((hi_sv & 16383) - PHALF).reshape(CAP_BLK, 1, EBP)
    nblo = (nlo + EBP - 1) // EBP
    nbhi = (nhi + EBP - 1) // EBP
    counts2 = jnp.stack([jnp.full((16,), nblo, jnp.int32),
                         jnp.full((16,), nbhi, jnp.int32)])

    sigp = jnp.concatenate([signals.reshape(N),
                            jnp.zeros((A1_ROWS - N,), jnp.float32)])

    w0row = W0.reshape(1, FEAT)
    b0row = b0.reshape(1, FEAT)
    grow = gn_gamma.reshape(1, FEAT)
    berow = gn_beta.reshape(1, FEAT)
    alrow = gn_alpha.reshape(1, FEAT)
    zrow = jnp.zeros((1, FEAT), jnp.float32)

    t1 = _agg1(sigp, srcp, dstp).reshape(A1_ROWS, 1)
    stats = _tc_stats(t1, w0row, b0row)
    x3 = _tc_norm(t1, stats, w0row, b0row, grow, berow, alrow)

    layer_params = [
        (W1_0.T, b1_0.reshape(1, FEAT), W2_0.T, b2_0.reshape(1, FEAT)),
        (W1_1.T, b1_1.reshape(1, FEAT), W2_1.T, zrow),
        (W1_2.T, b1_2.reshape(1, FEAT), W2_2.T, zrow),
    ]
    for w1t, b1row, w2t, b2row in layer_params:
        h3 = _aggp(x3, lo_src, lo_dst, hi_src, hi_dst, counts2)
        x3 = _tc_mlp(h3, x3, w1t, b1row, w2t, b2row)

    return _tc_final(x3, Wf.T, bf.reshape(1, NOUT))


# speedup vs baseline: 2.1927x; 2.1927x over previous
import dataclasses
import functools

import jax
import jax.numpy as jnp
from jax import lax
from jax.experimental import pallas as pl
from jax.experimental.pallas import tpu as pltpu
from jax.experimental.pallas import tpu_sc as plsc

N = 10000
FEAT = 256
HALF = 128
NOUT = 128
E = 160000

EB = 128
NBLK = 1280
EP = NBLK * EB
NSUB = 16
BPS = NBLK // NSUB
A1_ROWS = 10112
RED = 640

PHALF = N // 2
PACC = PHALF + 8
EBP = 64
CAP_BLK = 1408
CAP = CAP_BLK * EBP
QMAX = CAP_BLK // NSUB
QCH = QMAX // 2
RPP = 312
PTAIL = PHALF - NSUB * RPP

BM = 400
GRID = N // BM

_PREC = lax.Precision.DEFAULT

_mesh = plsc.VectorSubcoreMesh(core_axis_name="c", subcore_axis_name="s")

_sc_params = pltpu.CompilerParams()
if "needs_layout_passes" in pltpu.CompilerParams.__dataclass_fields__:
    _sc_params = dataclasses.replace(_sc_params, needs_layout_passes=False)


def _agg1_body(sigp_hbm, srcb_hbm, dstb_hbm, out_hbm,
               sig_v, idx3s, idx3d, acc_v, red_v, outacc, sh, sem):
    cid = lax.axis_index("c")
    sid = lax.axis_index("s")

    @pl.when(cid == 0)
    def _():
        pltpu.sync_copy(sigp_hbm, sig_v)
        pltpu.sync_copy(srcb_hbm.at[pl.ds(sid * BPS, BPS)], idx3s)
        pltpu.sync_copy(dstb_hbm.at[pl.ds(sid * BPS, BPS)], idx3d)

        @pl.loop(0, A1_ROWS // 16)
        def _(i):
            acc_v[pl.ds(i * 16, 16)] = jnp.zeros((16,), jnp.float32)

        @pl.loop(0, BPS)
        def _(j):
            @pl.loop(0, EB // 16)
            def _(k):
                sv = idx3s[j, 0, pl.ds(k * 16, 16)]
                dv = idx3d[j, 0, pl.ds(k * 16, 16)]
                vals = plsc.load_gather(sig_v, [sv])
                plsc.addupdate_scatter(acc_v, [dv], vals)

        pltpu.sync_copy(acc_v, sh.at[sid, 0])
        plsc.subcore_barrier()

        c0 = jnp.minimum(sid * RED, A1_ROWS - RED)
        pltpu.sync_copy(sh.at[:, :, pl.ds(c0, RED)], red_v)

        @pl.loop(0, RED // 16)
        def _(i):
            v = sig_v[pl.ds(c0 + i * 16, 16)]
            for k in range(NSUB):
                v = v + red_v[k, 0, pl.ds(i * 16, 16)]
            outacc[pl.ds(i * 16, 16)] = v

        pltpu.sync_copy(outacc, out_hbm.at[pl.ds(c0, RED)])


@functools.partial(
    pl.kernel,
    mesh=_mesh,
    out_type=jax.ShapeDtypeStruct((A1_ROWS,), jnp.float32),
    scratch_types=[
        pltpu.VMEM((A1_ROWS,), jnp.float32),
        pltpu.VMEM((BPS, 1, EB), jnp.int32),
        pltpu.VMEM((BPS, 1, EB), jnp.int32),
        pltpu.VMEM((A1_ROWS,), jnp.float32),
        pltpu.VMEM((NSUB, 1, RED), jnp.float32),
        pltpu.VMEM((RED,), jnp.float32),
        pltpu.VMEM_SHARED((NSUB, 1, A1_ROWS), jnp.float32),
        pltpu.SemaphoreType.DMA,
    ],
    compiler_params=_sc_params,
)
def _agg1(*args):
    _agg1_body(*args)


def _aggp_body(x3_hbm, losrc, lodst, hisrc, hidst, cnt2_hbm, out_hbm,
               srcv, dstv, cntv, rowsbuf, acc, semg0, semg1):
    cid = lax.axis_index("c")
    sid = lax.axis_index("s")
    base = cid * PHALF
    semg = (semg0, semg1)

    pltpu.sync_copy(x3_hbm.at[pl.ds(base + sid * RPP, RPP)],
                    acc.at[pl.ds(sid * RPP, RPP)])

    @pl.when(sid == NSUB - 1)
    def _():
        pltpu.sync_copy(x3_hbm.at[pl.ds(base + NSUB * RPP, PTAIL)],
                        acc.at[pl.ds(NSUB * RPP, PTAIL)])

    pltpu.sync_copy(cnt2_hbm, cntv)
    nb0 = jnp.max(cntv[0, pl.ds(0, 16)])
    nb1 = jnp.max(cntv[1, pl.ds(0, 16)])
    nb = jnp.where(cid == 0, nb0, nb1)
    quota = (nb + NSUB - 1) // NSUB
    s0 = sid * quota
    myn = jnp.clip(nb - s0, 0, quota)

    plsc.subcore_barrier()

    def g_slot(hp):
        return rowsbuf.at[pl.ds(hp * EBP, EBP)]

    def run_class(sb, db):
        for ch in range(2):
            off0 = s0 + ch * QCH
            n_ch = jnp.clip(myn - ch * QCH, 0, QCH)

            @pl.when(n_ch > 0)
            def _():
                pltpu.sync_copy(sb.at[pl.ds(off0, QCH)], srcv)
                pltpu.sync_copy(db.at[pl.ds(off0, QCH)], dstv)

                pltpu.make_async_copy(x3_hbm.at[srcv.at[0, 0]], g_slot(0),
                                      semg[0]).start()

                @pl.when(n_ch > 1)
                def _():
                    pltpu.make_async_copy(x3_hbm.at[srcv.at[1, 0]], g_slot(1),
                                          semg[1]).start()

                @pl.loop(0, (n_ch + 1) // 2)
                def _(g2):
                    for hp in range(2):
                        j = g2 * 2 + hp

                        @pl.when(j < n_ch)
                        def _():
                            pltpu.make_async_copy(x3_hbm.at[srcv.at[0, 0]],
                                                  g_slot(hp), semg[hp]).wait()

                            @pl.when(j + 2 < n_ch)
                            def _():
                                pltpu.make_async_copy(
                                    x3_hbm.at[srcv.at[j + 2, 0]],
                                    g_slot(hp), semg[hp]).start()

                            pltpu.sync_copy(g_slot(hp),
                                            acc.at[dstv.at[j, 0]], add=True)

    @pl.when(cid == 0)
    def _():
        run_class(losrc, lodst)

    @pl.when(cid == 1)
    def _():
        run_class(hisrc, hidst)

    plsc.subcore_barrier()

    pltpu.sync_copy(acc.at[pl.ds(sid * RPP, RPP)],
                    out_hbm.at[pl.ds(base + sid * RPP, RPP)])

    @pl.when(sid == NSUB - 1)
    def _():
        pltpu.sync_copy(acc.at[pl.ds(NSUB * RPP, PTAIL)],
                        out_hbm.at[pl.ds(base + NSUB * RPP, PTAIL)])


@functools.partial(
    pl.kernel,
    mesh=_mesh,
    out_type=jax.ShapeDtypeStruct((N, 2, HALF), jnp.float32),
    scratch_types=[
        pltpu.VMEM((QCH, 1, EBP), jnp.int32),
        pltpu.VMEM((QCH, 1, EBP), jnp.int32),
        pltpu.VMEM((2, 16), jnp.int32),
        pltpu.VMEM((2 * EBP, 2, HALF), jnp.float32),
        pltpu.VMEM_SHARED((PACC, 2, HALF), jnp.float32),
        pltpu.SemaphoreType.DMA,
        pltpu.SemaphoreType.DMA,
    ],
    compiler_params=_sc_params,
)
def _aggp(*args):
    _aggp_body(*args)


def _stats_body(t_ref, w0_ref, b0_ref, stats_ref):
    i = pl.program_id(0)
    xp = jnp.maximum(t_ref[...] * w0_ref[...] + b0_ref[...], 0.0)

    @pl.when(i == 0)
    def _():
        stats_ref[...] = jnp.zeros_like(stats_ref)

    stats_ref[0:1, :] += jnp.sum(xp, axis=0, keepdims=True)
    stats_ref[1:2, :] += jnp.sum(xp * xp, axis=0, keepdims=True)


def _norm_body(t_ref, stats_ref, w0_ref, b0_ref, g_ref, be_ref, al_ref, out_ref):
    xp = jnp.maximum(t_ref[...] * w0_ref[...] + b0_ref[...], 0.0)
    m = stats_ref[0:1, :] * (1.0 / N)
    ex2 = stats_ref[1:2, :] * (1.0 / N)
    al = al_ref[...]
    var = ex2 - 2.0 * al * m * m + al * al * m * m
    y = g_ref[...] * ((xp - al * m) * lax.rsqrt(var + 1e-5)) + be_ref[...]
    out_ref[...] = y.reshape(BM, 2, HALF)


def _mlp_body(h_ref, x_ref, w1_ref, b1_ref, w2_ref, b2_ref, out_ref):
    h = h_ref[...].reshape(BM, FEAT)
    z = jnp.maximum(
        jnp.dot(h, w1_ref[...], preferred_element_type=jnp.float32,
                precision=_PREC) + b1_ref[...], 0.0)
    y = jnp.dot(z, w2_ref[...], preferred_element_type=jnp.float32,
                precision=_PREC) + b2_ref[...]
    x = x_ref[...].reshape(BM, FEAT)
    xn = x + jnp.maximum(y, 0.0)
    out_ref[...] = xn.reshape(BM, 2, HALF)


def _mlpf_body(h_ref, x_ref, w1_ref, b1_ref, w2_ref, wf_ref, bf_ref, out_ref):
    h = h_ref[...].reshape(BM, FEAT)
    z = jnp.maximum(
        jnp.dot(h, w1_ref[...], preferred_element_type=jnp.float32,
                precision=_PREC) + b1_ref[...], 0.0)
    y = jnp.dot(z, w2_ref[...], preferred_element_type=jnp.float32,
                precision=_PREC)
    x = x_ref[...].reshape(BM, FEAT)
    xn = x + jnp.maximum(y, 0.0)
    out_ref[...] = jnp.dot(xn, wf_ref[...], preferred_element_type=jnp.float32,
                           precision=_PREC) + bf_ref[...]


def _row_spec(shape):
    return pl.BlockSpec(shape, lambda i: tuple(0 for _ in shape))


def _tc_stats(t1, w0row, b0row):
    return pl.pallas_call(
        _stats_body,
        grid=(GRID,),
        in_specs=[
            pl.BlockSpec((BM, 1), lambda i: (i, 0)),
            _row_spec((1, FEAT)),
            _row_spec((1, FEAT)),
        ],
        out_specs=pl.BlockSpec((8, FEAT), lambda i: (0, 0)),
        out_shape=jax.ShapeDtypeStruct((8, FEAT), jnp.float32),
    )(t1, w0row, b0row)


def _tc_norm(t1, stats, w0row, b0row, grow, berow, alrow):
    return pl.pallas_call(
        _norm_body,
        grid=(GRID,),
        in_specs=[
            pl.BlockSpec((BM, 1), lambda i: (i, 0)),
            _row_spec((8, FEAT)),
            _row_spec((1, FEAT)),
            _row_spec((1, FEAT)),
            _row_spec((1, FEAT)),
            _row_spec((1, FEAT)),
            _row_spec((1, FEAT)),
        ],
        out_specs=pl.BlockSpec((BM, 2, HALF), lambda i: (i, 0, 0)),
        out_shape=jax.ShapeDtypeStruct((N, 2, HALF), jnp.float32),
    )(t1, stats, w0row, b0row, grow, berow, alrow)


def _tc_mlp(h3, x3, w1t, b1row, w2t, b2row):
    return pl.pallas_call(
        _mlp_body,
        grid=(GRID,),
        in_specs=[
            pl.BlockSpec((BM, 2, HALF), lambda i: (i, 0, 0)),
            pl.BlockSpec((BM, 2, HALF), lambda i: (i, 0, 0)),
            _row_spec((FEAT, FEAT)),
            _row_spec((1, FEAT)),
            _row_spec((FEAT, FEAT)),
            _row_spec((1, FEAT)),
        ],
        out_specs=pl.BlockSpec((BM, 2, HALF), lambda i: (i, 0, 0)),
        out_shape=jax.ShapeDtypeStruct((N, 2, HALF), jnp.float32),
    )(h3, x3, w1t, b1row, w2t, b2row)


def _tc_mlpf(h3, x3, w1t, b1row, w2t, wft, bfrow):
    return pl.pallas_call(
        _mlpf_body,
        grid=(GRID,),
        in_specs=[
            pl.BlockSpec((BM, 2, HALF), lambda i: (i, 0, 0)),
            pl.BlockSpec((BM, 2, HALF), lambda i: (i, 0, 0)),
            _row_spec((FEAT, FEAT)),
            _row_spec((1, FEAT)),
            _row_spec((FEAT, FEAT)),
            _row_spec((FEAT, NOUT)),
            _row_spec((1, NOUT)),
        ],
        out_specs=pl.BlockSpec((BM, NOUT), lambda i: (i, 0)),
        out_shape=jax.ShapeDtypeStruct((N, NOUT), jnp.float32),
    )(h3, x3, w1t, b1row, w2t, wft, bfrow)


def kernel(signals, edge_index, W0, b0, W1_0, b1_0, W2_0, b2_0, W1_1, b1_1,
           W2_1, W1_2, b1_2, W2_2, gn_gamma, gn_beta, gn_alpha, Wf, bf):
    src = edge_index[0].astype(jnp.int32)
    dst = edge_index[1].astype(jnp.int32)
    pad = EP - E
    srcp = jnp.concatenate([src, jnp.zeros((pad,), jnp.int32)]).reshape(NBLK, 1, EB)
    dstp = jnp.concatenate([dst, jnp.full((pad,), N, jnp.int32)]).reshape(NBLK, 1, EB)

    key = (dst >= PHALF).astype(jnp.int32)
    sv = (key << 28) | (src << 14) | dst
    svs = jnp.sort(sv)
    nlo = E - jnp.sum(key)
    nhi = E - nlo
    ar = jnp.arange(CAP, dtype=jnp.int32)
    pad_hi = (1 << 28) | N
    svp = jnp.concatenate([svs, jnp.full((CAP,), pad_hi, jnp.int32)])
    lo_sv = jnp.where(ar < nlo, svs[:CAP], PHALF)
    hi_sv = lax.dynamic_slice(svp, (nlo,), (CAP,))
    lo_src = ((lo_sv >> 14) & 16383).reshape(CAP_BLK, 1, EBP)
    lo_dst = (lo_sv & 16383).reshape(CAP_BLK, 1, EBP)
    hi_src = ((hi_sv >> 14) & 16383).reshape(CAP_BLK, 1, EBP)
    hi_dst = ((hi_sv & 16383) - PHALF).reshape(CAP_BLK, 1, EBP)
    nblo = (nlo + EBP - 1) // EBP
    nbhi = (nhi + EBP - 1) // EBP
    counts2 = jnp.stack([jnp.full((16,), nblo, jnp.int32),
                         jnp.full((16,), nbhi, jnp.int32)])

    sigp = jnp.concatenate([signals.reshape(N),
                            jnp.zeros((A1_ROWS - N,), jnp.float32)])

    w0row = W0.reshape(1, FEAT)
    b0row = b0.reshape(1, FEAT)
    grow = gn_gamma.reshape(1, FEAT)
    berow = gn_beta.reshape(1, FEAT)
    alrow = gn_alpha.reshape(1, FEAT)
    zrow = jnp.zeros((1, FEAT), jnp.float32)

    t1 = _agg1(sigp, srcp, dstp).reshape(A1_ROWS, 1)
    stats = _tc_stats(t1, w0row, b0row)
    x3 = _tc_norm(t1, stats, w0row, b0row, grow, berow, alrow)

    layer_params = [
        (W1_0.T, b1_0.reshape(1, FEAT), W2_0.T, b2_0.reshape(1, FEAT)),
        (W1_1.T, b1_1.reshape(1, FEAT), W2_1.T, zrow),
    ]
    for w1t, b1row, w2t, b2row in layer_params:
        h3 = _aggp(x3, lo_src, lo_dst, hi_src, hi_dst, counts2)
        x3 = _tc_mlp(h3, x3, w1t, b1row, w2t, b2row)

    h3 = _aggp(x3, lo_src, lo_dst, hi_src, hi_dst, counts2)
    return _tc_mlpf(h3, x3, W1_2.T, b1_2.reshape(1, FEAT), W2_2.T,
                    Wf.T, bf.reshape(1, NOUT))


# speedup vs baseline: 2.6921x; 1.2277x over previous
import dataclasses
import functools

import jax
import jax.numpy as jnp
from jax import lax
from jax.experimental import pallas as pl
from jax.experimental.pallas import tpu as pltpu
from jax.experimental.pallas import tpu_sc as plsc

N = 10000
FEAT = 256
HALF = 128
NOUT = 128
E = 160000

EB = 128
NBLK = 1280
EP = NBLK * EB
NSUB = 16
BPS = NBLK // NSUB
A1_ROWS = 10112
RED = 640

PHALF = N // 2
PACC = PHALF + 8
EBP = 64
CAP_BLK = 1408
CAP = CAP_BLK * EBP
QMAX = CAP_BLK // NSUB
QCH = QMAX // 2
RPP = 312
PTAIL = PHALF - NSUB * RPP

BM = 400
GRID = N // BM

_PREC = lax.Precision.DEFAULT

_mesh = plsc.VectorSubcoreMesh(core_axis_name="c", subcore_axis_name="s")

_sc_params = pltpu.CompilerParams()
if "needs_layout_passes" in pltpu.CompilerParams.__dataclass_fields__:
    _sc_params = dataclasses.replace(_sc_params, needs_layout_passes=False)


def _agg1_body(sigp_hbm, srcb_hbm, dstb_hbm, out_hbm,
               sig_v, idx3s, idx3d, acc_v, red_v, outacc, sh, sem):
    cid = lax.axis_index("c")
    sid = lax.axis_index("s")

    @pl.when(cid == 0)
    def _():
        pltpu.sync_copy(sigp_hbm, sig_v)
        pltpu.sync_copy(srcb_hbm.at[pl.ds(sid * BPS, BPS)], idx3s)
        pltpu.sync_copy(dstb_hbm.at[pl.ds(sid * BPS, BPS)], idx3d)

        @pl.loop(0, A1_ROWS // 16)
        def _(i):
            acc_v[pl.ds(i * 16, 16)] = jnp.zeros((16,), jnp.float32)

        @pl.loop(0, BPS)
        def _(j):
            @pl.loop(0, EB // 16)
            def _(k):
                sv = idx3s[j, 0, pl.ds(k * 16, 16)]
                dv = idx3d[j, 0, pl.ds(k * 16, 16)]
                vals = plsc.load_gather(sig_v, [sv])
                plsc.addupdate_scatter(acc_v, [dv], vals)

        pltpu.sync_copy(acc_v, sh.at[sid, 0])
        plsc.subcore_barrier()

        c0 = jnp.minimum(sid * RED, A1_ROWS - RED)
        pltpu.sync_copy(sh.at[:, :, pl.ds(c0, RED)], red_v)

        @pl.loop(0, RED // 16)
        def _(i):
            v = sig_v[pl.ds(c0 + i * 16, 16)]
            for k in range(NSUB):
                v = v + red_v[k, 0, pl.ds(i * 16, 16)]
            outacc[pl.ds(i * 16, 16)] = v

        pltpu.sync_copy(outacc, out_hbm.at[pl.ds(c0, RED)])


@functools.partial(
    pl.kernel,
    mesh=_mesh,
    out_type=jax.ShapeDtypeStruct((A1_ROWS,), jnp.float32),
    scratch_types=[
        pltpu.VMEM((A1_ROWS,), jnp.float32),
        pltpu.VMEM((BPS, 1, EB), jnp.int32),
        pltpu.VMEM((BPS, 1, EB), jnp.int32),
        pltpu.VMEM((A1_ROWS,), jnp.float32),
        pltpu.VMEM((NSUB, 1, RED), jnp.float32),
        pltpu.VMEM((RED,), jnp.float32),
        pltpu.VMEM_SHARED((NSUB, 1, A1_ROWS), jnp.float32),
        pltpu.SemaphoreType.DMA,
    ],
    compiler_params=_sc_params,
)
def _agg1(*args):
    _agg1_body(*args)


def _aggp_body(x3_hbm, losrc, lodst, hisrc, hidst, cnt2_hbm, out_hbm,
               srcv, dstv, cntv, rowsbuf, acc, semg0, semg1):
    cid = lax.axis_index("c")
    sid = lax.axis_index("s")
    base = cid * PHALF
    semg = (semg0, semg1)

    pltpu.sync_copy(x3_hbm.at[pl.ds(base + sid * RPP, RPP)],
                    acc.at[pl.ds(sid * RPP, RPP)])

    @pl.when(sid == NSUB - 1)
    def _():
        pltpu.sync_copy(x3_hbm.at[pl.ds(base + NSUB * RPP, PTAIL)],
                        acc.at[pl.ds(NSUB * RPP, PTAIL)])

    pltpu.sync_copy(cnt2_hbm, cntv)
    nb0 = jnp.max(cntv[0, pl.ds(0, 16)])
    nb1 = jnp.max(cntv[1, pl.ds(0, 16)])
    nb = jnp.where(cid == 0, nb0, nb1)
    quota = (nb + NSUB - 1) // NSUB
    s0 = sid * quota
    myn = jnp.clip(nb - s0, 0, quota)

    plsc.subcore_barrier()

    def g_slot(hp):
        return rowsbuf.at[pl.ds(hp * EBP, EBP)]

    def run_class(sb, db):
        for ch in range(2):
            off0 = s0 + ch * QCH
            n_ch = jnp.clip(myn - ch * QCH, 0, QCH)

            @pl.when(n_ch > 0)
            def _():
                pltpu.sync_copy(sb.at[pl.ds(off0, QCH)], srcv)
                pltpu.sync_copy(db.at[pl.ds(off0, QCH)], dstv)

                pltpu.make_async_copy(x3_hbm.at[srcv.at[0, 0]], g_slot(0),
                                      semg[0]).start()

                @pl.when(n_ch > 1)
                def _():
                    pltpu.make_async_copy(x3_hbm.at[srcv.at[1, 0]], g_slot(1),
                                          semg[1]).start()

                @pl.loop(0, (n_ch + 1) // 2)
                def _(g2):
                    for hp in range(2):
                        j = g2 * 2 + hp

                        @pl.when(j < n_ch)
                        def _():
                            pltpu.make_async_copy(x3_hbm.at[srcv.at[0, 0]],
                                                  g_slot(hp), semg[hp]).wait()

                            @pl.when(j + 2 < n_ch)
                            def _():
                                pltpu.make_async_copy(
                                    x3_hbm.at[srcv.at[j + 2, 0]],
                                    g_slot(hp), semg[hp]).start()

                            pltpu.sync_copy(g_slot(hp),
                                            acc.at[dstv.at[j, 0]], add=True)

    @pl.when(cid == 0)
    def _():
        run_class(losrc, lodst)

    @pl.when(cid == 1)
    def _():
        run_class(hisrc, hidst)

    plsc.subcore_barrier()

    pltpu.sync_copy(acc.at[pl.ds(sid * RPP, RPP)],
                    out_hbm.at[pl.ds(base + sid * RPP, RPP)])

    @pl.when(sid == NSUB - 1)
    def _():
        pltpu.sync_copy(acc.at[pl.ds(NSUB * RPP, PTAIL)],
                        out_hbm.at[pl.ds(base + NSUB * RPP, PTAIL)])


@functools.partial(
    pl.kernel,
    mesh=_mesh,
    out_type=jax.ShapeDtypeStruct((N, 2, HALF), jnp.float32),
    scratch_types=[
        pltpu.VMEM((QCH, 1, EBP), jnp.int32),
        pltpu.VMEM((QCH, 1, EBP), jnp.int32),
        pltpu.VMEM((2, 16), jnp.int32),
        pltpu.VMEM((2 * EBP, 2, HALF), jnp.float32),
        pltpu.VMEM_SHARED((PACC, 2, HALF), jnp.float32),
        pltpu.SemaphoreType.DMA,
        pltpu.SemaphoreType.DMA,
    ],
    compiler_params=_sc_params,
)
def _aggp(*args):
    _aggp_body(*args)


def _stats_body(t_ref, w0_ref, b0_ref, stats_ref):
    i = pl.program_id(0)
    xp = jnp.maximum(t_ref[...] * w0_ref[...] + b0_ref[...], 0.0)

    @pl.when(i == 0)
    def _():
        stats_ref[...] = jnp.zeros_like(stats_ref)

    stats_ref[0:1, :] += jnp.sum(xp, axis=0, keepdims=True)
    stats_ref[1:2, :] += jnp.sum(xp * xp, axis=0, keepdims=True)


def _norm_body(t_ref, stats_ref, w0_ref, b0_ref, g_ref, be_ref, al_ref, out_ref):
    xp = jnp.maximum(t_ref[...] * w0_ref[...] + b0_ref[...], 0.0)
    m = stats_ref[0:1, :] * (1.0 / N)
    ex2 = stats_ref[1:2, :] * (1.0 / N)
    al = al_ref[...]
    var = ex2 - 2.0 * al * m * m + al * al * m * m
    y = g_ref[...] * ((xp - al * m) * lax.rsqrt(var + 1e-5)) + be_ref[...]
    out_ref[...] = y.reshape(BM, 2, HALF)


def _mlp_body(h_ref, x_ref, w1_ref, b1_ref, w2_ref, b2_ref, out_ref):
    h = h_ref[...].reshape(BM, FEAT)
    z = jnp.maximum(
        jnp.dot(h, w1_ref[...], preferred_element_type=jnp.float32,
                precision=_PREC) + b1_ref[...], 0.0)
    y = jnp.dot(z, w2_ref[...], preferred_element_type=jnp.float32,
                precision=_PREC) + b2_ref[...]
    x = x_ref[...].reshape(BM, FEAT)
    xn = x + jnp.maximum(y, 0.0)
    out_ref[...] = xn.reshape(BM, 2, HALF)


def _mlpf_body(h_ref, x_ref, w1_ref, b1_ref, w2_ref, wf_ref, bf_ref, out_ref):
    h = h_ref[...].reshape(BM, FEAT)
    z = jnp.maximum(
        jnp.dot(h, w1_ref[...], preferred_element_type=jnp.float32,
                precision=_PREC) + b1_ref[...], 0.0)
    y = jnp.dot(z, w2_ref[...], preferred_element_type=jnp.float32,
                precision=_PREC)
    x = x_ref[...].reshape(BM, FEAT)
    xn = x + jnp.maximum(y, 0.0)
    out_ref[...] = jnp.dot(xn, wf_ref[...], preferred_element_type=jnp.float32,
                           precision=_PREC) + bf_ref[...]


def _row_spec(shape):
    return pl.BlockSpec(shape, lambda i: tuple(0 for _ in shape))


def _tc_stats(t1, w0row, b0row):
    return pl.pallas_call(
        _stats_body,
        grid=(GRID,),
        in_specs=[
            pl.BlockSpec((BM, 1), lambda i: (i, 0)),
            _row_spec((1, FEAT)),
            _row_spec((1, FEAT)),
        ],
        out_specs=pl.BlockSpec((8, FEAT), lambda i: (0, 0)),
        out_shape=jax.ShapeDtypeStruct((8, FEAT), jnp.float32),
    )(t1, w0row, b0row)


def _tc_norm(t1, stats, w0row, b0row, grow, berow, alrow):
    return pl.pallas_call(
        _norm_body,
        grid=(GRID,),
        in_specs=[
            pl.BlockSpec((BM, 1), lambda i: (i, 0)),
            _row_spec((8, FEAT)),
            _row_spec((1, FEAT)),
            _row_spec((1, FEAT)),
            _row_spec((1, FEAT)),
            _row_spec((1, FEAT)),
            _row_spec((1, FEAT)),
        ],
        out_specs=pl.BlockSpec((BM, 2, HALF), lambda i: (i, 0, 0)),
        out_shape=jax.ShapeDtypeStruct((N, 2, HALF), jnp.float32),
    )(t1, stats, w0row, b0row, grow, berow, alrow)


def _tc_mlp(h3, x3, w1t, b1row, w2t, b2row):
    return pl.pallas_call(
        _mlp_body,
        grid=(GRID,),
        in_specs=[
            pl.BlockSpec((BM, 2, HALF), lambda i: (i, 0, 0)),
            pl.BlockSpec((BM, 2, HALF), lambda i: (i, 0, 0)),
            _row_spec((FEAT, FEAT)),
            _row_spec((1, FEAT)),
            _row_spec((FEAT, FEAT)),
            _row_spec((1, FEAT)),
        ],
        out_specs=pl.BlockSpec((BM, 2, HALF), lambda i: (i, 0, 0)),
        out_shape=jax.ShapeDtypeStruct((N, 2, HALF), jnp.float32),
    )(h3, x3, w1t, b1row, w2t, b2row)


def _tc_mlpf(h3, x3, w1t, b1row, w2t, wft, bfrow):
    return pl.pallas_call(
        _mlpf_body,
        grid=(GRID,),
        in_specs=[
            pl.BlockSpec((BM, 2, HALF), lambda i: (i, 0, 0)),
            pl.BlockSpec((BM, 2, HALF), lambda i: (i, 0, 0)),
            _row_spec((FEAT, FEAT)),
            _row_spec((1, FEAT)),
            _row_spec((FEAT, FEAT)),
            _row_spec((FEAT, NOUT)),
            _row_spec((1, NOUT)),
        ],
        out_specs=pl.BlockSpec((BM, NOUT), lambda i: (i, 0)),
        out_shape=jax.ShapeDtypeStruct((N, NOUT), jnp.float32),
    )(h3, x3, w1t, b1row, w2t, wft, bfrow)


def kernel(signals, edge_index, W0, b0, W1_0, b1_0, W2_0, b2_0, W1_1, b1_1,
           W2_1, W1_2, b1_2, W2_2, gn_gamma, gn_beta, gn_alpha, Wf, bf):
    src = edge_index[0].astype(jnp.int32)
    dst = edge_index[1].astype(jnp.int32)
    pad = EP - E
    srcp = jnp.concatenate([src, jnp.zeros((pad,), jnp.int32)]).reshape(NBLK, 1, EB)
    dstp = jnp.concatenate([dst, jnp.full((pad,), N, jnp.int32)]).reshape(NBLK, 1, EB)

    key = (dst >= PHALF).astype(jnp.int32)
    sv = (key << 28) | (src << 14) | dst
    svs = lax.sort(sv, is_stable=False)
    nlo = E - jnp.sum(key)
    nhi = E - nlo
    ar = jnp.arange(CAP, dtype=jnp.int32)
    pad_hi = (1 << 28) | N
    svp = jnp.concatenate([svs, jnp.full((CAP,), pad_hi, jnp.int32)])
    lo_sv = jnp.where(ar < nlo, svs[:CAP], PHALF)
    hi_sv = lax.dynamic_slice(svp, (nlo,), (CAP,))
    lo_src = ((lo_sv >> 14) & 16383).reshape(CAP_BLK, 1, EBP)
    lo_dst = (lo_sv & 16383).reshape(CAP_BLK, 1, EBP)
    hi_src = ((hi_sv >> 14) & 16383).reshape(CAP_BLK, 1, EBP)
    hi_dst = ((hi_sv & 16383) - PHALF).reshape(CAP_BLK, 1, EBP)
    nblo = (nlo + EBP - 1) // EBP
    nbhi = (nhi + EBP - 1) // EBP
    counts2 = jnp.stack([jnp.full((16,), nblo, jnp.int32),
                         jnp.full((16,), nbhi, jnp.int32)])

    sigp = jnp.concatenate([signals.reshape(N),
                            jnp.zeros((A1_ROWS - N,), jnp.float32)])

    w0row = W0.reshape(1, FEAT)
    b0row = b0.reshape(1, FEAT)
    grow = gn_gamma.reshape(1, FEAT)
    berow = gn_beta.reshape(1, FEAT)
    alrow = gn_alpha.reshape(1, FEAT)
    zrow = jnp.zeros((1, FEAT), jnp.float32)

    t1 = _agg1(sigp, srcp, dstp).reshape(A1_ROWS, 1)
    stats = _tc_stats(t1, w0row, b0row)
    x3 = _tc_norm(t1, stats, w0row, b0row, grow, berow, alrow)

    layer_params = [
        (W1_0.T, b1_0.reshape(1, FEAT), W2_0.T, b2_0.reshape(1, FEAT)),
        (W1_1.T, b1_1.reshape(1, FEAT), W2_1.T, zrow),
    ]
    for w1t, b1row, w2t, b2row in layer_params:
        h3 = _aggp(x3, lo_src, lo_dst, hi_src, hi_dst, counts2)
        x3 = _tc_mlp(h3, x3, w1t, b1row, w2t, b2row)

    h3 = _aggp(x3, lo_src, lo_dst, hi_src, hi_dst, counts2)
    return _tc_mlpf(h3, x3, W1_2.T, b1_2.reshape(1, FEAT), W2_2.T,
                    Wf.T, bf.reshape(1, NOUT))


# speedup vs baseline: 2.7378x; 1.0170x over previous
import dataclasses
import functools

import jax
import jax.numpy as jnp
from jax import lax
from jax.experimental import pallas as pl
from jax.experimental.pallas import tpu as pltpu
from jax.experimental.pallas import tpu_sc as plsc

N = 10000
FEAT = 256
HALF = 128
NOUT = 128
E = 160000

EB = 128
NBLK = 1280
EP = NBLK * EB
NSUB = 16
BPS = NBLK // NSUB
A1_ROWS = 10112
RED = 640

PHALF = N // 2
PACC = PHALF + 8
EBP = 64
CAP_BLK = 1408
CAP = CAP_BLK * EBP
QMAX = CAP_BLK // NSUB
QCH = QMAX // 2
RPP = 312
PTAIL = PHALF - NSUB * RPP

BM = 400
GRID = N // BM

_PREC = lax.Precision.DEFAULT

_mesh = plsc.VectorSubcoreMesh(core_axis_name="c", subcore_axis_name="s")

_sc_params = pltpu.CompilerParams()
if "needs_layout_passes" in pltpu.CompilerParams.__dataclass_fields__:
    _sc_params = dataclasses.replace(_sc_params, needs_layout_passes=False)


def _agg1_body(sigp_hbm, srcb_hbm, dstb_hbm, out_hbm,
               sig_v, idx3s, idx3d, acc_v, red_v, outacc, sh, sem):
    cid = lax.axis_index("c")
    sid = lax.axis_index("s")

    @pl.when(cid == 0)
    def _():
        pltpu.sync_copy(sigp_hbm, sig_v)
        pltpu.sync_copy(srcb_hbm.at[pl.ds(sid * BPS, BPS)], idx3s)
        pltpu.sync_copy(dstb_hbm.at[pl.ds(sid * BPS, BPS)], idx3d)

        @pl.loop(0, A1_ROWS // 16)
        def _(i):
            acc_v[pl.ds(i * 16, 16)] = jnp.zeros((16,), jnp.float32)

        @pl.loop(0, BPS)
        def _(j):
            @pl.loop(0, EB // 16)
            def _(k):
                sv = idx3s[j, 0, pl.ds(k * 16, 16)]
                dv = idx3d[j, 0, pl.ds(k * 16, 16)]
                vals = plsc.load_gather(sig_v, [sv])
                plsc.addupdate_scatter(acc_v, [dv], vals)

        pltpu.sync_copy(acc_v, sh.at[sid, 0])
        plsc.subcore_barrier()

        c0 = jnp.minimum(sid * RED, A1_ROWS - RED)
        pltpu.sync_copy(sh.at[:, :, pl.ds(c0, RED)], red_v)

        @pl.loop(0, RED // 16)
        def _(i):
            v = sig_v[pl.ds(c0 + i * 16, 16)]
            for k in range(NSUB):
                v = v + red_v[k, 0, pl.ds(i * 16, 16)]
            outacc[pl.ds(i * 16, 16)] = v

        pltpu.sync_copy(outacc, out_hbm.at[pl.ds(c0, RED)])


@functools.partial(
    pl.kernel,
    mesh=_mesh,
    out_type=jax.ShapeDtypeStruct((A1_ROWS,), jnp.float32),
    scratch_types=[
        pltpu.VMEM((A1_ROWS,), jnp.float32),
        pltpu.VMEM((BPS, 1, EB), jnp.int32),
        pltpu.VMEM((BPS, 1, EB), jnp.int32),
        pltpu.VMEM((A1_ROWS,), jnp.float32),
        pltpu.VMEM((NSUB, 1, RED), jnp.float32),
        pltpu.VMEM((RED,), jnp.float32),
        pltpu.VMEM_SHARED((NSUB, 1, A1_ROWS), jnp.float32),
        pltpu.SemaphoreType.DMA,
    ],
    compiler_params=_sc_params,
)
def _agg1(*args):
    _agg1_body(*args)


def _aggp_body(x3_hbm, losrc, lodst, hisrc, hidst, cnt2_hbm, out_hbm,
               srcv, dstv, cntv, rowsbuf, acc, semg0, semg1):
    cid = lax.axis_index("c")
    sid = lax.axis_index("s")
    base = cid * PHALF
    semg = (semg0, semg1)

    pltpu.sync_copy(x3_hbm.at[pl.ds(base + sid * RPP, RPP)],
                    acc.at[pl.ds(sid * RPP, RPP)])

    @pl.when(sid == NSUB - 1)
    def _():
        pltpu.sync_copy(x3_hbm.at[pl.ds(base + NSUB * RPP, PTAIL)],
                        acc.at[pl.ds(NSUB * RPP, PTAIL)])

    pltpu.sync_copy(cnt2_hbm, cntv)
    nb0 = jnp.max(cntv[0, pl.ds(0, 16)])
    nb1 = jnp.max(cntv[1, pl.ds(0, 16)])
    nb = jnp.where(cid == 0, nb0, nb1)
    quota = (nb + NSUB - 1) // NSUB
    s0 = sid * quota
    myn = jnp.clip(nb - s0, 0, quota)

    def g_slot(hp):
        return rowsbuf.at[pl.ds(hp * EBP, EBP)]

    def stage_chunk(sb, db, off0, n_ch):
        pltpu.sync_copy(sb.at[pl.ds(off0, QCH)], srcv)
        pltpu.sync_copy(db.at[pl.ds(off0, QCH)], dstv)

        pltpu.make_async_copy(x3_hbm.at[srcv.at[0, 0]], g_slot(0),
                              semg[0]).start()

        @pl.when(n_ch > 1)
        def _():
            pltpu.make_async_copy(x3_hbm.at[srcv.at[1, 0]], g_slot(1),
                                  semg[1]).start()

    def pre_class(sb, db):
        n0 = jnp.minimum(myn, QCH)

        @pl.when(n0 > 0)
        def _():
            stage_chunk(sb, db, s0, n0)

    def run_class(sb, db):
        for ch in range(2):
            off0 = s0 + ch * QCH
            n_ch = jnp.clip(myn - ch * QCH, 0, QCH)

            @pl.when(n_ch > 0)
            def _():
                if ch > 0:
                    stage_chunk(sb, db, off0, n_ch)

                @pl.loop(0, (n_ch + 1) // 2)
                def _(g2):
                    for hp in range(2):
                        j = g2 * 2 + hp

                        @pl.when(j < n_ch)
                        def _():
                            pltpu.make_async_copy(x3_hbm.at[srcv.at[0, 0]],
                                                  g_slot(hp), semg[hp]).wait()

                            @pl.when(j + 2 < n_ch)
                            def _():
                                pltpu.make_async_copy(
                                    x3_hbm.at[srcv.at[j + 2, 0]],
                                    g_slot(hp), semg[hp]).start()

                            pltpu.sync_copy(g_slot(hp),
                                            acc.at[dstv.at[j, 0]], add=True)

    @pl.when(cid == 0)
    def _():
        pre_class(losrc, lodst)

    @pl.when(cid == 1)
    def _():
        pre_class(hisrc, hidst)

    plsc.subcore_barrier()

    @pl.when(cid == 0)
    def _():
        run_class(losrc, lodst)

    @pl.when(cid == 1)
    def _():
        run_class(hisrc, hidst)

    plsc.subcore_barrier()

    pltpu.sync_copy(acc.at[pl.ds(sid * RPP, RPP)],
                    out_hbm.at[pl.ds(base + sid * RPP, RPP)])

    @pl.when(sid == NSUB - 1)
    def _():
        pltpu.sync_copy(acc.at[pl.ds(NSUB * RPP, PTAIL)],
                        out_hbm.at[pl.ds(base + NSUB * RPP, PTAIL)])


@functools.partial(
    pl.kernel,
    mesh=_mesh,
    out_type=jax.ShapeDtypeStruct((N, 2, HALF), jnp.float32),
    scratch_types=[
        pltpu.VMEM((QCH, 1, EBP), jnp.int32),
        pltpu.VMEM((QCH, 1, EBP), jnp.int32),
        pltpu.VMEM((2, 16), jnp.int32),
        pltpu.VMEM((2 * EBP, 2, HALF), jnp.float32),
        pltpu.VMEM_SHARED((PACC, 2, HALF), jnp.float32),
        pltpu.SemaphoreType.DMA,
        pltpu.SemaphoreType.DMA,
    ],
    compiler_params=_sc_params,
)
def _aggp(*args):
    _aggp_body(*args)


def _stats_body(t_ref, w0_ref, b0_ref, stats_ref):
    i = pl.program_id(0)
    xp = jnp.maximum(t_ref[...] * w0_ref[...] + b0_ref[...], 0.0)

    @pl.when(i == 0)
    def _():
        stats_ref[...] = jnp.zeros_like(stats_ref)

    stats_ref[0:1, :] += jnp.sum(xp, axis=0, keepdims=True)
    stats_ref[1:2, :] += jnp.sum(xp * xp, axis=0, keepdims=True)


def _norm_body(t_ref, stats_ref, w0_ref, b0_ref, g_ref, be_ref, al_ref, out_ref):
    xp = jnp.maximum(t_ref[...] * w0_ref[...] + b0_ref[...], 0.0)
    m = stats_ref[0:1, :] * (1.0 / N)
    ex2 = stats_ref[1:2, :] * (1.0 / N)
    al = al_ref[...]
    var = ex2 - 2.0 * al * m * m + al * al * m * m
    y = g_ref[...] * ((xp - al * m) * lax.rsqrt(var + 1e-5)) + be_ref[...]
    out_ref[...] = y.reshape(BM, 2, HALF)


def _mlp_body(h_ref, x_ref, w1_ref, b1_ref, w2_ref, b2_ref, out_ref):
    h = h_ref[...].reshape(BM, FEAT)
    z = jnp.maximum(
        jnp.dot(h, w1_ref[...], preferred_element_type=jnp.float32,
                precision=_PREC) + b1_ref[...], 0.0)
    y = jnp.dot(z, w2_ref[...], preferred_element_type=jnp.float32,
                precision=_PREC) + b2_ref[...]
    x = x_ref[...].reshape(BM, FEAT)
    xn = x + jnp.maximum(y, 0.0)
    out_ref[...] = xn.reshape(BM, 2, HALF)


def _mlpf_body(h_ref, x_ref, w1_ref, b1_ref, w2_ref, wf_ref, bf_ref, out_ref):
    h = h_ref[...].reshape(BM, FEAT)
    z = jnp.maximum(
        jnp.dot(h, w1_ref[...], preferred_element_type=jnp.float32,
                precision=_PREC) + b1_ref[...], 0.0)
    y = jnp.dot(z, w2_ref[...], preferred_element_type=jnp.float32,
                precision=_PREC)
    x = x_ref[...].reshape(BM, FEAT)
    xn = x + jnp.maximum(y, 0.0)
    out_ref[...] = jnp.dot(xn, wf_ref[...], preferred_element_type=jnp.float32,
                           precision=_PREC) + bf_ref[...]


def _row_spec(shape):
    return pl.BlockSpec(shape, lambda i: tuple(0 for _ in shape))


def _tc_stats(t1, w0row, b0row):
    return pl.pallas_call(
        _stats_body,
        grid=(GRID,),
        in_specs=[
            pl.BlockSpec((BM, 1), lambda i: (i, 0)),
            _row_spec((1, FEAT)),
            _row_spec((1, FEAT)),
        ],
        out_specs=pl.BlockSpec((8, FEAT), lambda i: (0, 0)),
        out_shape=jax.ShapeDtypeStruct((8, FEAT), jnp.float32),
    )(t1, w0row, b0row)


def _tc_norm(t1, stats, w0row, b0row, grow, berow, alrow):
    return pl.pallas_call(
        _norm_body,
        grid=(GRID,),
        in_specs=[
            pl.BlockSpec((BM, 1), lambda i: (i, 0)),
            _row_spec((8, FEAT)),
            _row_spec((1, FEAT)),
            _row_spec((1, FEAT)),
            _row_spec((1, FEAT)),
            _row_spec((1, FEAT)),
            _row_spec((1, FEAT)),
        ],
        out_specs=pl.BlockSpec((BM, 2, HALF), lambda i: (i, 0, 0)),
        out_shape=jax.ShapeDtypeStruct((N, 2, HALF), jnp.float32),
    )(t1, stats, w0row, b0row, grow, berow, alrow)


def _tc_mlp(h3, x3, w1t, b1row, w2t, b2row):
    return pl.pallas_call(
        _mlp_body,
        grid=(GRID,),
        in_specs=[
            pl.BlockSpec((BM, 2, HALF), lambda i: (i, 0, 0)),
            pl.BlockSpec((BM, 2, HALF), lambda i: (i, 0, 0)),
            _row_spec((FEAT, FEAT)),
            _row_spec((1, FEAT)),
            _row_spec((FEAT, FEAT)),
            _row_spec((1, FEAT)),
        ],
        out_specs=pl.BlockSpec((BM, 2, HALF), lambda i: (i, 0, 0)),
        out_shape=jax.ShapeDtypeStruct((N, 2, HALF), jnp.float32),
    )(h3, x3, w1t, b1row, w2t, b2row)


def _tc_mlpf(h3, x3, w1t, b1row, w2t, wft, bfrow):
    return pl.pallas_call(
        _mlpf_body,
        grid=(GRID,),
        in_specs=[
            pl.BlockSpec((BM, 2, HALF), lambda i: (i, 0, 0)),
            pl.BlockSpec((BM, 2, HALF), lambda i: (i, 0, 0)),
            _row_spec((FEAT, FEAT)),
            _row_spec((1, FEAT)),
            _row_spec((FEAT, FEAT)),
            _row_spec((FEAT, NOUT)),
            _row_spec((1, NOUT)),
        ],
        out_specs=pl.BlockSpec((BM, NOUT), lambda i: (i, 0)),
        out_shape=jax.ShapeDtypeStruct((N, NOUT), jnp.float32),
    )(h3, x3, w1t, b1row, w2t, wft, bfrow)


def kernel(signals, edge_index, W0, b0, W1_0, b1_0, W2_0, b2_0, W1_1, b1_1,
           W2_1, W1_2, b1_2, W2_2, gn_gamma, gn_beta, gn_alpha, Wf, bf):
    src = edge_index[0].astype(jnp.int32)
    dst = edge_index[1].astype(jnp.int32)
    pad = EP - E
    srcp = jnp.concatenate([src, jnp.zeros((pad,), jnp.int32)]).reshape(NBLK, 1, EB)
    dstp = jnp.concatenate([dst, jnp.full((pad,), N, jnp.int32)]).reshape(NBLK, 1, EB)

    key = (dst >= PHALF).astype(jnp.int32)
    sv = (key << 28) | (src << 14) | dst
    svs = lax.sort(sv, is_stable=False)
    nlo = E - jnp.sum(key)
    nhi = E - nlo
    ar = jnp.arange(CAP, dtype=jnp.int32)
    pad_hi = (1 << 28) | N
    svp = jnp.concatenate([svs, jnp.full((CAP,), pad_hi, jnp.int32)])
    lo_sv = jnp.where(ar < nlo, svs[:CAP], PHALF)
    hi_sv = lax.dynamic_slice(svp, (nlo,), (CAP,))
    lo_src = ((lo_sv >> 14) & 16383).reshape(CAP_BLK, 1, EBP)
    lo_dst = (lo_sv & 16383).reshape(CAP_BLK, 1, EBP)
    hi_src = ((hi_sv >> 14) & 16383).reshape(CAP_BLK, 1, EBP)
    hi_dst = ((hi_sv & 16383) - PHALF).reshape(CAP_BLK, 1, EBP)
    nblo = (nlo + EBP - 1) // EBP
    nbhi = (nhi + EBP - 1) // EBP
    counts2 = jnp.stack([jnp.full((16,), nblo, jnp.int32),
                         jnp.full((16,), nbhi, jnp.int32)])

    sigp = jnp.concatenate([signals.reshape(N),
                            jnp.zeros((A1_ROWS - N,), jnp.float32)])

    w0row = W0.reshape(1, FEAT)
    b0row = b0.reshape(1, FEAT)
    grow = gn_gamma.reshape(1, FEAT)
    berow = gn_beta.reshape(1, FEAT)
    alrow = gn_alpha.reshape(1, FEAT)
    zrow = jnp.zeros((1, FEAT), jnp.float32)

    t1 = _agg1(sigp, srcp, dstp).reshape(A1_ROWS, 1)
    stats = _tc_stats(t1, w0row, b0row)
    x3 = _tc_norm(t1, stats, w0row, b0row, grow, berow, alrow)

    layer_params = [
        (W1_0.T, b1_0.reshape(1, FEAT), W2_0.T, b2_0.reshape(1, FEAT)),
        (W1_1.T, b1_1.reshape(1, FEAT), W2_1.T, zrow),
    ]
    for w1t, b1row, w2t, b2row in layer_params:
        h3 = _aggp(x3, lo_src, lo_dst, hi_src, hi_dst, counts2)
        x3 = _tc_mlp(h3, x3, w1t, b1row, w2t, b2row)

    h3 = _aggp(x3, lo_src, lo_dst, hi_src, hi_dst, counts2)
    return _tc_mlpf(h3, x3, W1_2.T, b1_2.reshape(1, FEAT), W2_2.T,
                    Wf.T, bf.reshape(1, NOUT))
